# baseline (speedup 1.0000x reference)
import functools

import jax
import jax.numpy as jnp
from jax import lax
from jax.experimental import pallas as pl
from jax.experimental.pallas import tpu as pltpu
from jax.experimental.pallas import tpu_sc as plsc

R_MAX = 5.0
B_RMAX = 1.01
B_RMIN = -1.01
NUM_BASIS = 128
_LANES = 16


def _newton_rsqrt(x):
    i = plsc.bitcast(x, jnp.int32)
    i = jnp.int32(0x5F3759DF) - lax.shift_right_logical(i, 1)
    y = plsc.bitcast(i, jnp.float32)
    y = y * (1.5 - 0.5 * x * y * y)
    y = y * (1.5 - 0.5 * x * y * y)
    y = y * (1.5 - 0.5 * x * y * y)
    return y


def _sc_edge_scalars(pos_flat, spin_flat, row, col):
    n_edges = row.shape[0]
    n_words = pos_flat.shape[0]
    try:
        info = plsc.get_sparse_core_info()
        num_cores, num_subcores = info.num_cores, info.num_subcores
    except ValueError:
        num_cores, num_subcores = 2, 16
    nw = num_cores * num_subcores
    epw = n_edges // nw
    assert epw * nw == n_edges and epw % _LANES == 0 and epw % 8 == 0

    mesh = plsc.VectorSubcoreMesh(
        core_axis_name="c", subcore_axis_name="s",
        num_cores=num_cores, num_subcores=num_subcores)

    @functools.partial(
        pl.kernel,
        out_type=(
            jax.ShapeDtypeStruct((n_edges,), jnp.float32),
            jax.ShapeDtypeStruct((n_edges,), jnp.float32),
        ),
        mesh=mesh,
        compiler_params=pltpu.CompilerParams(needs_layout_passes=False),
        scratch_types=[
            pltpu.VMEM((n_words,), jnp.float32),
            pltpu.VMEM((n_words,), jnp.float32),
            pltpu.VMEM((epw,), jnp.int32),
            pltpu.VMEM((epw,), jnp.int32),
            pltpu.VMEM((epw,), jnp.float32),
            pltpu.VMEM((epw,), jnp.float32),
        ],
    )
    def sc_kernel(pos_hbm, spin_hbm, row_hbm, col_hbm, theta_hbm, scale_hbm,
                  pos_v, spin_v, row_v, col_v, theta_v, scale_v):
        wid = lax.axis_index("s") * num_cores + lax.axis_index("c")
        base = wid * epw
        pltpu.sync_copy(pos_hbm, pos_v)
        pltpu.sync_copy(spin_hbm, spin_v)
        pltpu.sync_copy(row_hbm.at[pl.ds(base, epw)], row_v)
        pltpu.sync_copy(col_hbm.at[pl.ds(base, epw)], col_v)

        def body(i, _):
            off = i * _LANES
            r3 = row_v[pl.ds(off, _LANES)] * 3
            c3 = col_v[pl.ds(off, _LANES)] * 3
            dx = plsc.load_gather(pos_v, [r3]) - plsc.load_gather(pos_v, [c3])
            dy = (plsc.load_gather(pos_v, [r3 + 1])
                  - plsc.load_gather(pos_v, [c3 + 1]))
            dz = (plsc.load_gather(pos_v, [r3 + 2])
                  - plsc.load_gather(pos_v, [c3 + 2]))
            sxr = plsc.load_gather(spin_v, [r3])
            syr = plsc.load_gather(spin_v, [r3 + 1])
            szr = plsc.load_gather(spin_v, [r3 + 2])
            sxc = plsc.load_gather(spin_v, [c3])
            syc = plsc.load_gather(spin_v, [c3 + 1])
            szc = plsc.load_gather(spin_v, [c3 + 2])

            d2 = dx * dx + dy * dy + dz * dz + 1e-12
            r_len = d2 * _newton_rsqrt(d2)
            x = r_len * (1.0 / R_MAX)
            x3 = x * x * x
            x6 = x3 * x3
            x7 = x6 * x
            x8 = x7 * x
            cut = 1.0 - 28.0 * x6 + 48.0 * x7 - 21.0 * x8
            cut = jnp.where(x < 1.0, cut, 0.0)

            sr2 = sxr * sxr + syr * syr + szr * szr
            sc2 = sxc * sxc + syc * syc + szc * szc
            snr = sr2 * _newton_rsqrt(sr2)
            snc = sc2 * _newton_rsqrt(sc2)
            dot = sxr * sxc + syr * syc + szr * szc
            sisj = dot / ((snr + 1e-9) * (snc + 1e-9))

            theta_v[pl.ds(off, _LANES)] = (
                (sisj - B_RMIN) * (0.5 / (B_RMAX - B_RMIN)))
            scale_v[pl.ds(off, _LANES)] = sr2 * sc2 * cut
            return ()

        lax.fori_loop(0, epw // _LANES, body, (), unroll=False)
        pltpu.sync_copy(theta_v, theta_hbm.at[pl.ds(base, epw)])
        pltpu.sync_copy(scale_v, scale_hbm.at[pl.ds(base, epw)])

    return sc_kernel(pos_flat, spin_flat, row, col)


_COS_POLY = (0.99999998906233145, -19.739204499762209, 64.939117467842323,
             -85.450139613558932, 60.167631326705617, -25.967599889485943,
             6.5286582592453151)
_RND_MAGIC = 12582912.0


def _tc_body(ef_ref, phsc_ref, out_ref):
    d_feat = ef_ref.shape[1]
    out_ref[:, :d_feat] = ef_ref[...]
    n = lax.broadcasted_iota(jnp.int32, (2, 2 * NUM_BASIS), 1).astype(
        jnp.float32)
    row = lax.broadcasted_iota(jnp.int32, (2, 2 * NUM_BASIS), 0)
    in_right = lax.broadcasted_iota(
        jnp.int32, (2, 2 * NUM_BASIS), 1) >= NUM_BASIS
    rhs = jnp.where(row == 0,
                    jnp.where(in_right, 0.0, n),
                    jnp.where(in_right, 1.0, 0.0))
    both = lax.dot_general(
        phsc_ref[0], rhs,
        dimension_numbers=(((0,), (0,)), ((), ())),
        precision=lax.Precision.HIGHEST,
        preferred_element_type=jnp.float32)
    t = both[:, :NUM_BASIS]
    sc_b = both[:, NUM_BASIS:]
    r = t - ((t + _RND_MAGIC) - _RND_MAGIC)
    s = r * r
    p = jnp.float32(_COS_POLY[6])
    for c in _COS_POLY[5::-1]:
        p = p * s + jnp.float32(c)
    out_ref[:, d_feat:] = p * sc_b


def _tc_assemble(edge_features, phi, scale):
    n_edges, d_feat = edge_features.shape
    be = 8000
    assert n_edges % be == 0
    grid = (n_edges // be,)
    phsc = jnp.stack(
        [phi.reshape(grid[0], be), scale.reshape(grid[0], be)], axis=1)
    return pl.pallas_call(
        _tc_body,
        grid=grid,
        in_specs=[
            pl.BlockSpec((be, d_feat), lambda i: (i, 0)),
            pl.BlockSpec((1, 2, be), lambda i: (i, 0, 0)),
        ],
        out_specs=pl.BlockSpec((be, d_feat + NUM_BASIS), lambda i: (i, 0)),
        out_shape=jax.ShapeDtypeStruct(
            (n_edges, d_feat + NUM_BASIS), jnp.float32),
        compiler_params=pltpu.CompilerParams(
            dimension_semantics=("arbitrary",)),
    )(edge_features, phsc)


def kernel(edge_features, edge_index, pos, spin):
    row = edge_index[0]
    col = edge_index[1]
    phi, scale = _sc_edge_scalars(
        pos.reshape(-1), spin.reshape(-1), row, col)
    return _tc_assemble(edge_features, phi, scale)


# speedup vs baseline: 1.6374x; 1.6374x over previous
import functools

import jax
import jax.numpy as jnp
from jax import lax
from jax.experimental import pallas as pl
from jax.experimental.pallas import tpu as pltpu
from jax.experimental.pallas import tpu_sc as plsc

R_MAX = 5.0
B_RMAX = 1.01
B_RMIN = -1.01
NUM_BASIS = 128
_LANES = 16


def _newton_rsqrt(x):
    i = plsc.bitcast(x, jnp.int32)
    i = jnp.int32(0x5F3759DF) - lax.shift_right_logical(i, 1)
    y = plsc.bitcast(i, jnp.float32)
    y = y * (1.5 - 0.5 * x * y * y)
    y = y * (1.5 - 0.5 * x * y * y)
    y = y * (1.5 - 0.5 * x * y * y)
    return y


def _sc_edge_scalars(pos_flat, spin_flat, row, col):
    n_edges = row.shape[0]
    n_words = pos_flat.shape[0]
    try:
        info = plsc.get_sparse_core_info()
        num_cores, num_subcores = info.num_cores, info.num_subcores
    except ValueError:
        num_cores, num_subcores = 2, 16
    nw = num_cores * num_subcores
    epw = n_edges // nw
    assert epw * nw == n_edges and epw % _LANES == 0 and epw % 8 == 0

    mesh = plsc.VectorSubcoreMesh(
        core_axis_name="c", subcore_axis_name="s",
        num_cores=num_cores, num_subcores=num_subcores)

    @functools.partial(
        pl.kernel,
        out_type=(
            jax.ShapeDtypeStruct((n_edges,), jnp.float32),
            jax.ShapeDtypeStruct((n_edges,), jnp.float32),
        ),
        mesh=mesh,
        compiler_params=pltpu.CompilerParams(needs_layout_passes=False),
        scratch_types=[
            pltpu.VMEM((n_words,), jnp.float32),
            pltpu.VMEM((n_words,), jnp.float32),
            pltpu.VMEM((epw,), jnp.int32),
            pltpu.VMEM((epw,), jnp.int32),
            pltpu.VMEM((epw,), jnp.float32),
            pltpu.VMEM((epw,), jnp.float32),
        ],
    )
    def sc_kernel(pos_hbm, spin_hbm, row_hbm, col_hbm, theta_hbm, scale_hbm,
                  pos_v, spin_v, row_v, col_v, theta_v, scale_v):
        wid = lax.axis_index("s") * num_cores + lax.axis_index("c")
        base = wid * epw
        pltpu.sync_copy(pos_hbm, pos_v)
        pltpu.sync_copy(spin_hbm, spin_v)
        pltpu.sync_copy(row_hbm.at[pl.ds(base, epw)], row_v)
        pltpu.sync_copy(col_hbm.at[pl.ds(base, epw)], col_v)

        def body(i, _):
            off = i * _LANES
            r3 = row_v[pl.ds(off, _LANES)] * 3
            c3 = col_v[pl.ds(off, _LANES)] * 3
            dx = plsc.load_gather(pos_v, [r3]) - plsc.load_gather(pos_v, [c3])
            dy = (plsc.load_gather(pos_v, [r3 + 1])
                  - plsc.load_gather(pos_v, [c3 + 1]))
            dz = (plsc.load_gather(pos_v, [r3 + 2])
                  - plsc.load_gather(pos_v, [c3 + 2]))
            sxr = plsc.load_gather(spin_v, [r3])
            syr = plsc.load_gather(spin_v, [r3 + 1])
            szr = plsc.load_gather(spin_v, [r3 + 2])
            sxc = plsc.load_gather(spin_v, [c3])
            syc = plsc.load_gather(spin_v, [c3 + 1])
            szc = plsc.load_gather(spin_v, [c3 + 2])

            d2 = dx * dx + dy * dy + dz * dz + 1e-12
            r_len = d2 * _newton_rsqrt(d2)
            x = r_len * (1.0 / R_MAX)
            x3 = x * x * x
            x6 = x3 * x3
            x7 = x6 * x
            x8 = x7 * x
            cut = 1.0 - 28.0 * x6 + 48.0 * x7 - 21.0 * x8
            cut = jnp.where(x < 1.0, cut, 0.0)

            sr2 = sxr * sxr + syr * syr + szr * szr
            sc2 = sxc * sxc + syc * syc + szc * szc
            snr = sr2 * _newton_rsqrt(sr2)
            snc = sc2 * _newton_rsqrt(sc2)
            dot = sxr * sxc + syr * syc + szr * szc
            sisj = dot / ((snr + 1e-9) * (snc + 1e-9))

            theta_v[pl.ds(off, _LANES)] = (
                (sisj - B_RMIN) * (0.5 / (B_RMAX - B_RMIN)))
            scale_v[pl.ds(off, _LANES)] = sr2 * sc2 * cut
            return ()

        lax.fori_loop(0, epw // _LANES, body, (), unroll=False)
        pltpu.sync_copy(theta_v, theta_hbm.at[pl.ds(base, epw)])
        pltpu.sync_copy(scale_v, scale_hbm.at[pl.ds(base, epw)])

    return sc_kernel(pos_flat, spin_flat, row, col)


_COS_POLY = (0.99999998906233145, -19.739204499762209, 64.939117467842323,
             -85.450139613558932, 60.167631326705617, -25.967599889485943,
             6.5286582592453151)
_RND_MAGIC = 12582912.0


def _tc_body(ef_ref, phsc_ref, out_ref):
    d_feat = ef_ref.shape[1]
    out_ref[:, :d_feat] = ef_ref[...]
    ph_sc = phsc_ref[0]
    hi = ph_sc.astype(jnp.bfloat16)
    r1 = ph_sc - hi.astype(jnp.float32)
    mid = r1.astype(jnp.bfloat16)
    lo = (r1 - mid.astype(jnp.float32)).astype(jnp.bfloat16)
    lhs = jnp.concatenate([hi, mid, lo], axis=0)
    colidx = lax.broadcasted_iota(jnp.int32, (6, 2 * NUM_BASIS), 1)
    rowidx = lax.broadcasted_iota(jnp.int32, (6, 2 * NUM_BASIS), 0)
    in_right = colidx >= NUM_BASIS
    rhs = jnp.where(rowidx % 2 == 0,
                    jnp.where(in_right, 0.0, colidx.astype(jnp.float32)),
                    jnp.where(in_right, 1.0, 0.0)).astype(jnp.bfloat16)
    both = lax.dot_general(
        lhs, rhs,
        dimension_numbers=(((0,), (0,)), ((), ())),
        preferred_element_type=jnp.float32)
    t = both[:, :NUM_BASIS]
    sc_b = both[:, NUM_BASIS:]
    r = t - ((t + _RND_MAGIC) - _RND_MAGIC)
    s = r * r
    p = jnp.float32(_COS_POLY[6])
    for c in _COS_POLY[5::-1]:
        p = p * s + jnp.float32(c)
    out_ref[:, d_feat:] = p * sc_b


def _tc_assemble(edge_features, phi, scale):
    n_edges, d_feat = edge_features.shape
    be = 8000
    assert n_edges % be == 0
    grid = (n_edges // be,)
    phsc = jnp.stack(
        [phi.reshape(grid[0], be), scale.reshape(grid[0], be)], axis=1)
    return pl.pallas_call(
        _tc_body,
        grid=grid,
        in_specs=[
            pl.BlockSpec((be, d_feat), lambda i: (i, 0)),
            pl.BlockSpec((1, 2, be), lambda i: (i, 0, 0)),
        ],
        out_specs=pl.BlockSpec((be, d_feat + NUM_BASIS), lambda i: (i, 0)),
        out_shape=jax.ShapeDtypeStruct(
            (n_edges, d_feat + NUM_BASIS), jnp.float32),
        compiler_params=pltpu.CompilerParams(
            dimension_semantics=("arbitrary",)),
    )(edge_features, phsc)


def kernel(edge_features, edge_index, pos, spin):
    row = edge_index[0]
    col = edge_index[1]
    phi, scale = _sc_edge_scalars(
        pos.reshape(-1), spin.reshape(-1), row, col)
    return _tc_assemble(edge_features, phi, scale)
